# baseline (speedup 1.0000x reference)
import jax
import jax.numpy as jnp
from jax import lax
from jax.experimental import pallas as pl
from jax.experimental.pallas import tpu as pltpu
from jax.experimental.pallas import tpu_sc as plsc

_N = 100000
_D = 128
_H = 10000
_GS = 32

_NC, _NS = 2, 16
_NW = _NC * _NS
_HPW = 320
_HPAD = _NW * _HPW
_CH = 4
_NCHUNK = _HPW // _CH
_ROWS = _CH * _GS
_NV = _D // 16
_NBUF = 2


def _mm_body(ne_ref, wt_ref, b_ref, x_ref):
    x_ref[...] = jnp.maximum(
        jnp.dot(ne_ref[...], wt_ref[...], preferred_element_type=jnp.float32)
        + b_ref[...], 0.0)


def _transform(ne, wt, b):
    bn = 1000
    return pl.pallas_call(
        _mm_body,
        grid=(_N // bn,),
        in_specs=[
            pl.BlockSpec((bn, _D), lambda i: (i, 0)),
            pl.BlockSpec((_D, _D), lambda i: (0, 0)),
            pl.BlockSpec((1, _D), lambda i: (0, 0)),
        ],
        out_specs=pl.BlockSpec((bn, _D), lambda i: (i, 0)),
        out_shape=jax.ShapeDtypeStruct((_N, _D), jnp.float32),
    )(ne, wt, b.reshape(1, _D))


def _sc_body(x_hbm, idx_hbm, out_hbm, idx_v, buf, out_v, sem0, sem1):
    wid = lax.axis_index("s") * _NC + lax.axis_index("c")
    sems = (sem0, sem1)
    pltpu.sync_copy(idx_hbm.at[wid], idx_v)

    for b in range(_NBUF):
        pltpu.async_copy(x_hbm.at[idx_v.at[b]], buf.at[b], sems[b])

    def reduce_chunk(c, b):
        for h in range(_CH):
            base = h * _GS
            row = c * _CH + h
            for d in range(_NV):
                off = d * 16
                vals = [buf[b, base + r, pl.ds(off, 16)] for r in range(_GS)]
                while len(vals) > 1:
                    vals = [vals[i] + vals[i + 1]
                            for i in range(0, len(vals), 2)]
                out_v[row, pl.ds(off, 16)] = vals[0] * (1.0 / _GS)

    def group(g, carry):
        for b in range(_NBUF):
            c = g * _NBUF + b
            pltpu.make_async_copy(
                x_hbm.at[idx_v.at[c]], buf.at[b], sems[b]).wait()
            reduce_chunk(c, b)
            pltpu.async_copy(
                x_hbm.at[idx_v.at[c + _NBUF]], buf.at[b], sems[b])
        return carry

    lax.fori_loop(0, _NCHUNK // _NBUF, group, 0)
    for b in range(_NBUF):
        pltpu.make_async_copy(
            x_hbm.at[idx_v.at[_NCHUNK + b]], buf.at[b], sems[b]).wait()
    pltpu.sync_copy(out_v, out_hbm.at[pl.ds(wid * _HPW, _HPW)])


def _aggregate(x, idx):
    mesh = plsc.VectorSubcoreMesh(core_axis_name="c", subcore_axis_name="s")
    k = pl.kernel(
        _sc_body,
        out_type=jax.ShapeDtypeStruct((_HPAD, _D), jnp.float32),
        mesh=mesh,
        scratch_types=[
            pltpu.VMEM((_NCHUNK + _NBUF, _ROWS), jnp.int32),
            pltpu.VMEM((_NBUF, _ROWS, _D), jnp.float32),
            pltpu.VMEM((_HPW, _D), jnp.float32),
            pltpu.SemaphoreType.DMA,
            pltpu.SemaphoreType.DMA,
        ],
    )
    return k(x, idx)


def kernel(node_embeddings, hyperedges, hyperedge_subsets, W, b):
    del hyperedges
    x = _transform(node_embeddings, W.T, b)
    idx = hyperedge_subsets.astype(jnp.int32).reshape(_H, _GS)
    idx = jnp.pad(idx, ((0, _HPAD - _H), (0, 0)))
    idx = idx.reshape(_NW, _NCHUNK, _ROWS)
    idx = jnp.pad(idx, ((0, 0), (0, _NBUF), (0, 0)))
    return _aggregate(x, idx)[:_H]


# speedup vs baseline: 1.2890x; 1.2890x over previous
import jax
import jax.numpy as jnp
from jax import lax
from jax.experimental import pallas as pl
from jax.experimental.pallas import tpu as pltpu
from jax.experimental.pallas import tpu_sc as plsc

_N = 100000
_D = 128
_H = 10000
_GS = 32

_NC, _NS = 2, 16
_NW = _NC * _NS
_HPW = 320
_HPAD = _NW * _HPW
_CH = 4
_NCHUNK = _HPW // _CH
_ROWS = _CH * _GS
_NV = _D // 16
_NBUF = 2


def _mm_body(ne_ref, wt_ref, b_ref, x_ref):
    x_ref[...] = jnp.maximum(
        jnp.dot(ne_ref[...], wt_ref[...], preferred_element_type=jnp.float32)
        + b_ref[...], 0.0)


def _transform(ne, wt, b):
    bn = 1000
    return pl.pallas_call(
        _mm_body,
        grid=(_N // bn,),
        in_specs=[
            pl.BlockSpec((bn, _D), lambda i: (i, 0)),
            pl.BlockSpec((_D, _D), lambda i: (0, 0)),
            pl.BlockSpec((1, _D), lambda i: (0, 0)),
        ],
        out_specs=pl.BlockSpec((bn, _D), lambda i: (i, 0)),
        out_shape=jax.ShapeDtypeStruct((_N, _D), jnp.float32),
    )(ne, wt, b.reshape(1, _D))


def _sc_body(x_hbm, idx_hbm, out_hbm, idx_v, buf, out_v, sem0, sem1):
    wid = lax.axis_index("s") * _NC + lax.axis_index("c")
    sems = (sem0, sem1)
    pltpu.sync_copy(idx_hbm.at[wid], idx_v)

    def reduce_chunk(c, b):
        for h in range(_CH):
            base = h * _GS
            row = c * _CH + h
            for d in range(_NV):
                off = d * 16
                vals = [buf[b, base + r, pl.ds(off, 16)] for r in range(_GS)]
                while len(vals) > 1:
                    vals = [vals[i] + vals[i + 1]
                            for i in range(0, len(vals), 2)]
                out_v[row, pl.ds(off, 16)] = vals[0] * (1.0 / _GS)

    def group(g, carry):
        for b in range(_NBUF):
            c = g * _NBUF + b
            pltpu.async_copy(x_hbm.at[idx_v.at[c]], buf.at[b], sems[b]).wait()
            reduce_chunk(c, b)
        return carry

    lax.fori_loop(0, _NCHUNK // _NBUF, group, 0)
    pltpu.sync_copy(out_v, out_hbm.at[pl.ds(wid * _HPW, _HPW)])


def _aggregate(x, idx):
    mesh = plsc.VectorSubcoreMesh(core_axis_name="c", subcore_axis_name="s")
    k = pl.kernel(
        _sc_body,
        out_type=jax.ShapeDtypeStruct((_HPAD, _D), jnp.float32),
        mesh=mesh,
        scratch_types=[
            pltpu.VMEM((_NCHUNK + _NBUF, _ROWS), jnp.int32),
            pltpu.VMEM((_NBUF, _ROWS, _D), jnp.float32),
            pltpu.VMEM((_HPW, _D), jnp.float32),
            pltpu.SemaphoreType.DMA,
            pltpu.SemaphoreType.DMA,
        ],
    )
    return k(x, idx)


def kernel(node_embeddings, hyperedges, hyperedge_subsets, W, b):
    del hyperedges
    x = _transform(node_embeddings, W.T, b)
    idx = hyperedge_subsets.astype(jnp.int32).reshape(_H, _GS)
    idx = jnp.pad(idx, ((0, _HPAD - _H), (0, 0)))
    idx = idx.reshape(_NW, _NCHUNK, _ROWS)
    idx = jnp.pad(idx, ((0, 0), (0, _NBUF), (0, 0)))
    return _aggregate(x, idx)[:_H]
